# baseline (speedup 1.0000x reference)
import functools

import jax
import jax.numpy as jnp
from jax import lax
from jax.experimental import pallas as pl
from jax.experimental.pallas import tpu as pltpu
from jax.experimental.pallas import tpu_sc as plsc

B, S, H, D = 4, 4096, 8, 64
NC, NS, L = 2, 16, 16
NW = NC * NS
TOTAL = B * S * H
BPW = TOTAL // NW
CHUNK = 128
NCHUNK = BPW // CHUNK


def _sc_gather(ids_flat, table, offs16):
    mesh = plsc.VectorSubcoreMesh(core_axis_name="c", subcore_axis_name="s")

    @functools.partial(
        pl.kernel,
        mesh=mesh,
        out_type=jax.ShapeDtypeStruct((TOTAL, D), jnp.float32),
        scratch_types=[
            pltpu.VMEM((BPW,), jnp.int32),
            pltpu.VMEM((L,), jnp.int32),
            pltpu.VMEM((CHUNK, D), jnp.float32),
            pltpu.SemaphoreType.DMA,
        ],
        compiler_params=pltpu.CompilerParams(use_tc_tiling_on_sc=False),
    )
    def k(ids_hbm, table_hbm, offs_hbm, out_hbm, idx_v, off_v, rows_v, gsem):
        wid = lax.axis_index("s") * NC + lax.axis_index("c")
        base = wid * BPW
        pltpu.sync_copy(ids_hbm.at[pl.ds(base, BPW)], idx_v)
        pltpu.sync_copy(offs_hbm, off_v)
        off = off_v[...]

        def add_body(i, carry):
            sl = pl.ds(i * L, L)
            idx_v[sl] = idx_v[sl] + off
            return carry

        lax.fori_loop(0, BPW // L, add_body, 0)

        def chunk_body(c, carry):
            pltpu.async_copy(
                table_hbm.at[idx_v.at[pl.ds(c * CHUNK, CHUNK)]], rows_v, gsem
            ).wait()
            pltpu.sync_copy(rows_v, out_hbm.at[pl.ds(base + c * CHUNK, CHUNK)])
            return carry

        lax.fori_loop(0, NCHUNK, chunk_body, 0)

    return k(ids_flat, table, offs16)


def kernel(input_ids, table, offsets):
    ids = input_ids.astype(jnp.int32).reshape(TOTAL)
    offs16 = jnp.tile(offsets.astype(jnp.int32), L // H)
    out = _sc_gather(ids, table, offs16)
    return out.reshape(B, S, H, D)


# speedup vs baseline: 1.0252x; 1.0252x over previous
import functools

import jax
import jax.numpy as jnp
from jax import lax
from jax.experimental import pallas as pl
from jax.experimental.pallas import tpu as pltpu
from jax.experimental.pallas import tpu_sc as plsc

B, S, H, D = 4, 4096, 8, 64
NC, NS, L = 2, 16, 16
NW = NC * NS
TOTAL = B * S * H
BPW = TOTAL // NW
CHUNK = 128
NCHUNK = BPW // CHUNK
NBUF = 2


def _sc_gather(ids_flat, table, offs16):
    mesh = plsc.VectorSubcoreMesh(core_axis_name="c", subcore_axis_name="s")

    @functools.partial(
        pl.kernel,
        mesh=mesh,
        out_type=jax.ShapeDtypeStruct((TOTAL, D), jnp.float32),
        scratch_types=[
            pltpu.VMEM((BPW,), jnp.int32),
            pltpu.VMEM((L,), jnp.int32),
            pltpu.VMEM((NBUF, CHUNK, D), jnp.float32),
            pltpu.SemaphoreType.DMA,
            pltpu.SemaphoreType.DMA,
            pltpu.SemaphoreType.DMA,
            pltpu.SemaphoreType.DMA,
        ],
        compiler_params=pltpu.CompilerParams(use_tc_tiling_on_sc=False),
    )
    def k(ids_hbm, table_hbm, offs_hbm, out_hbm, idx_v, off_v, rows_v,
          g0, g1, w0, w1):
        gsem = (g0, g1)
        wsem = (w0, w1)
        wid = lax.axis_index("s") * NC + lax.axis_index("c")
        base = wid * BPW
        pltpu.sync_copy(ids_hbm.at[pl.ds(base, BPW)], idx_v)
        pltpu.sync_copy(offs_hbm, off_v)
        off = off_v[...]

        def add_body(i, carry):
            sl = pl.ds(i * L, L)
            idx_v[sl] = idx_v[sl] + off
            return carry

        lax.fori_loop(0, BPW // L, add_body, 0)

        def start_gather(c, b):
            pltpu.async_copy(
                table_hbm.at[idx_v.at[pl.ds(c * CHUNK, CHUNK)]],
                rows_v.at[b], gsem[b],
            )

        def wait_gather(c, b):
            pltpu.make_async_copy(
                table_hbm.at[idx_v.at[pl.ds(c * CHUNK, CHUNK)]],
                rows_v.at[b], gsem[b],
            ).wait()

        def start_write(c, b):
            pltpu.async_copy(
                rows_v.at[b], out_hbm.at[pl.ds(base + c * CHUNK, CHUNK)],
                wsem[b],
            )

        def wait_write(c, b):
            pltpu.make_async_copy(
                rows_v.at[b], out_hbm.at[pl.ds(base + c * CHUNK, CHUNK)],
                wsem[b],
            ).wait()

        start_gather(0, 0)

        def ring_body(g2, carry):
            for b in range(NBUF):
                c = g2 * NBUF + b
                nb = (b + 1) % NBUF

                @pl.when(c >= 1)
                def _():
                    wait_write(c - 1, nb)

                @pl.when(c + 1 < NCHUNK)
                def _():
                    start_gather(c + 1, nb)

                wait_gather(c, b)
                start_write(c, b)
            return carry

        lax.fori_loop(0, NCHUNK // NBUF, ring_body, 0)
        wait_write(NCHUNK - 1, (NCHUNK - 1) % NBUF)

    return k(ids_flat, table, offs16)


def kernel(input_ids, table, offsets):
    ids = input_ids.astype(jnp.int32).reshape(TOTAL)
    offs16 = jnp.tile(offsets.astype(jnp.int32), L // H)
    out = _sc_gather(ids, table, offs16)
    return out.reshape(B, S, H, D)


# speedup vs baseline: 1.1850x; 1.1559x over previous
import functools

import jax
import jax.numpy as jnp
from jax import lax
from jax.experimental import pallas as pl
from jax.experimental.pallas import tpu as pltpu
from jax.experimental.pallas import tpu_sc as plsc

B, S, H, D = 4, 4096, 8, 64
DP = 128
TOTAL_ROWS = 800000
NC, NS, L = 2, 16, 16
NW = NC * NS
TOTAL = B * S * H
BPW = TOTAL // NW
CHUNK = 128
NCHUNK = BPW // CHUNK
NBUF = 2


def _sc_gather(ids_flat, table_pad, offs16):
    mesh = plsc.VectorSubcoreMesh(core_axis_name="c", subcore_axis_name="s")

    @functools.partial(
        pl.kernel,
        mesh=mesh,
        out_type=jax.ShapeDtypeStruct((TOTAL, DP), jnp.float32),
        scratch_types=[
            pltpu.VMEM((BPW,), jnp.int32),
            pltpu.VMEM((L,), jnp.int32),
            pltpu.VMEM((NBUF, CHUNK, DP), jnp.float32),
            pltpu.SemaphoreType.DMA,
            pltpu.SemaphoreType.DMA,
            pltpu.SemaphoreType.DMA,
            pltpu.SemaphoreType.DMA,
        ],
        compiler_params=pltpu.CompilerParams(use_tc_tiling_on_sc=False),
    )
    def k(ids_hbm, table_hbm, offs_hbm, out_hbm, idx_v, off_v, rows_v,
          g0, g1, w0, w1):
        gsem = (g0, g1)
        wsem = (w0, w1)
        wid = lax.axis_index("s") * NC + lax.axis_index("c")
        base = wid * BPW
        pltpu.sync_copy(ids_hbm.at[pl.ds(base, BPW)], idx_v)
        pltpu.sync_copy(offs_hbm, off_v)
        off = off_v[...]

        def add_body(i, carry):
            sl = pl.ds(i * L, L)
            idx_v[sl] = idx_v[sl] + off
            return carry

        lax.fori_loop(0, BPW // L, add_body, 0)

        def start_gather(c, b):
            pltpu.async_copy(
                table_hbm.at[idx_v.at[pl.ds(c * CHUNK, CHUNK)]],
                rows_v.at[b], gsem[b],
            )

        def wait_gather(c, b):
            pltpu.make_async_copy(
                table_hbm.at[idx_v.at[pl.ds(c * CHUNK, CHUNK)]],
                rows_v.at[b], gsem[b],
            ).wait()

        def start_write(c, b):
            pltpu.async_copy(
                rows_v.at[b], out_hbm.at[pl.ds(base + c * CHUNK, CHUNK)],
                wsem[b],
            )

        def wait_write(c, b):
            pltpu.make_async_copy(
                rows_v.at[b], out_hbm.at[pl.ds(base + c * CHUNK, CHUNK)],
                wsem[b],
            ).wait()

        start_gather(0, 0)

        def ring_body(g2, carry):
            for b in range(NBUF):
                c = g2 * NBUF + b
                nb = (b + 1) % NBUF

                @pl.when(c >= 1)
                def _():
                    wait_write(c - 1, nb)

                @pl.when(c + 1 < NCHUNK)
                def _():
                    start_gather(c + 1, nb)

                wait_gather(c, b)
                start_write(c, b)
            return carry

        lax.fori_loop(0, NCHUNK // NBUF, ring_body, 0)
        wait_write(NCHUNK - 1, (NCHUNK - 1) % NBUF)

    return k(ids_flat, table_pad, offs16)


def kernel(input_ids, table, offsets):
    ids = input_ids.astype(jnp.int32).reshape(TOTAL)
    offs16 = jnp.tile(offsets.astype(jnp.int32), L // H)
    table_pad = jnp.concatenate(
        [table, jnp.zeros((TOTAL_ROWS, DP - D), jnp.float32)], axis=1)
    out_pad = _sc_gather(ids, table_pad, offs16)
    return out_pad.reshape(B, S, H, DP)[..., :D]


# speedup vs baseline: 1.1895x; 1.0038x over previous
import functools

import jax
import jax.numpy as jnp
from jax import lax
from jax.experimental import pallas as pl
from jax.experimental.pallas import tpu as pltpu
from jax.experimental.pallas import tpu_sc as plsc

B, S, H, D = 4, 4096, 8, 64
DP = 128
TOTAL_ROWS = 800000
NC, NS, L = 2, 16, 16
NW = NC * NS
TOTAL = B * S * H
BPW = TOTAL // NW
CHUNK = 128
NCHUNK = BPW // CHUNK
NBUF = 2


def _sc_gather(ids_flat, table_pad, offs16):
    mesh = plsc.VectorSubcoreMesh(core_axis_name="c", subcore_axis_name="s")

    @functools.partial(
        pl.kernel,
        mesh=mesh,
        out_type=jax.ShapeDtypeStruct((TOTAL, DP), jnp.float32),
        scratch_types=[
            pltpu.VMEM((BPW,), jnp.int32),
            pltpu.VMEM((L,), jnp.int32),
            pltpu.VMEM((NBUF, CHUNK, DP), jnp.float32),
            pltpu.SemaphoreType.DMA,
            pltpu.SemaphoreType.DMA,
            pltpu.SemaphoreType.DMA,
            pltpu.SemaphoreType.DMA,
        ],
        compiler_params=pltpu.CompilerParams(use_tc_tiling_on_sc=False),
    )
    def k(ids_hbm, table_hbm, offs_hbm, out_hbm, idx_v, off_v, rows_v,
          g0, g1, w0, w1):
        gsem = (g0, g1)
        wsem = (w0, w1)
        wid = lax.axis_index("s") * NC + lax.axis_index("c")
        base = wid * BPW
        pltpu.sync_copy(ids_hbm.at[pl.ds(base, BPW)], idx_v)
        pltpu.sync_copy(offs_hbm, off_v)
        off = off_v[...]

        def add_body(i, carry):
            sl = pl.ds(i * L, L)
            idx_v[sl] = idx_v[sl] + off
            return carry

        lax.fori_loop(0, BPW // L, add_body, 0)

        def start_gather(c, b):
            pltpu.async_copy(
                table_hbm.at[idx_v.at[pl.ds(c * CHUNK, CHUNK)]],
                rows_v.at[b], gsem[b],
            )

        def wait_gather(c, b):
            pltpu.make_async_copy(
                table_hbm.at[idx_v.at[pl.ds(c * CHUNK, CHUNK)]],
                rows_v.at[b], gsem[b],
            ).wait()

        def start_write(c, b):
            pltpu.async_copy(
                rows_v.at[b], out_hbm.at[pl.ds(base + c * CHUNK, CHUNK)],
                wsem[b],
            )

        def wait_write(c, b):
            pltpu.make_async_copy(
                rows_v.at[b], out_hbm.at[pl.ds(base + c * CHUNK, CHUNK)],
                wsem[b],
            ).wait()

        for j in range(NBUF - 1):
            start_gather(j, j)

        def ring_body(g2, carry):
            for b in range(NBUF):
                c = g2 * NBUF + b
                pb = (b - 1) % NBUF

                @pl.when(c + NBUF - 1 < NCHUNK)
                def _():
                    @pl.when(c >= 1)
                    def _():
                        wait_write(c - 1, pb)

                    start_gather(c + NBUF - 1, pb)

                wait_gather(c, b)
                start_write(c, b)
            return carry

        lax.fori_loop(0, NCHUNK // NBUF, ring_body, 0)
        for i in range(NBUF):
            c = NCHUNK - NBUF + i
            wait_write(c, c % NBUF)

    return k(ids_flat, table_pad, offs16)


def kernel(input_ids, table, offsets):
    ids = input_ids.astype(jnp.int32).reshape(TOTAL)
    offs16 = jnp.tile(offsets.astype(jnp.int32), L // H)
    table_pad = jnp.concatenate(
        [table, jnp.zeros((TOTAL_ROWS, DP - D), jnp.float32)], axis=1)
    out_pad = _sc_gather(ids, table_pad, offs16)
    return out_pad.reshape(B, S, H, DP)[..., :D]


# speedup vs baseline: 1.1910x; 1.0012x over previous
import functools

import jax
import jax.numpy as jnp
from jax import lax
from jax.experimental import pallas as pl
from jax.experimental.pallas import tpu as pltpu
from jax.experimental.pallas import tpu_sc as plsc

B, S, H, D = 4, 4096, 8, 64
DP = 128
TOTAL_ROWS = 800000
NC, NS, L = 2, 16, 16
NW = NC * NS
TOTAL = B * S * H
BPW = TOTAL // NW
CHUNK = 256
NCHUNK = BPW // CHUNK
NBUF = 2


def _sc_gather(ids_flat, table_pad, offs16):
    mesh = plsc.VectorSubcoreMesh(core_axis_name="c", subcore_axis_name="s")

    @functools.partial(
        pl.kernel,
        mesh=mesh,
        out_type=jax.ShapeDtypeStruct((TOTAL, DP), jnp.float32),
        scratch_types=[
            pltpu.VMEM((BPW,), jnp.int32),
            pltpu.VMEM((L,), jnp.int32),
            pltpu.VMEM((NBUF, CHUNK, DP), jnp.float32),
            pltpu.SemaphoreType.DMA,
            pltpu.SemaphoreType.DMA,
            pltpu.SemaphoreType.DMA,
            pltpu.SemaphoreType.DMA,
        ],
        compiler_params=pltpu.CompilerParams(use_tc_tiling_on_sc=False),
    )
    def k(ids_hbm, table_hbm, offs_hbm, out_hbm, idx_v, off_v, rows_v,
          g0, g1, w0, w1):
        gsem = (g0, g1)
        wsem = (w0, w1)
        wid = lax.axis_index("s") * NC + lax.axis_index("c")
        base = wid * BPW
        pltpu.sync_copy(ids_hbm.at[pl.ds(base, BPW)], idx_v)
        pltpu.sync_copy(offs_hbm, off_v)
        off = off_v[...]

        def add_body(i, carry):
            sl = pl.ds(i * L, L)
            idx_v[sl] = idx_v[sl] + off
            return carry

        lax.fori_loop(0, BPW // L, add_body, 0)

        def start_gather(c, b):
            pltpu.async_copy(
                table_hbm.at[idx_v.at[pl.ds(c * CHUNK, CHUNK)]],
                rows_v.at[b], gsem[b],
            )

        def wait_gather(c, b):
            pltpu.make_async_copy(
                table_hbm.at[idx_v.at[pl.ds(c * CHUNK, CHUNK)]],
                rows_v.at[b], gsem[b],
            ).wait()

        def start_write(c, b):
            pltpu.async_copy(
                rows_v.at[b], out_hbm.at[pl.ds(base + c * CHUNK, CHUNK)],
                wsem[b],
            )

        def wait_write(c, b):
            pltpu.make_async_copy(
                rows_v.at[b], out_hbm.at[pl.ds(base + c * CHUNK, CHUNK)],
                wsem[b],
            ).wait()

        for j in range(NBUF - 1):
            start_gather(j, j)

        def ring_body(g2, carry):
            for b in range(NBUF):
                c = g2 * NBUF + b
                pb = (b - 1) % NBUF

                @pl.when(c + NBUF - 1 < NCHUNK)
                def _():
                    @pl.when(c >= 1)
                    def _():
                        wait_write(c - 1, pb)

                    start_gather(c + NBUF - 1, pb)

                wait_gather(c, b)
                start_write(c, b)
            return carry

        lax.fori_loop(0, NCHUNK // NBUF, ring_body, 0)
        for i in range(NBUF):
            c = NCHUNK - NBUF + i
            wait_write(c, c % NBUF)

    return k(ids_flat, table_pad, offs16)


def kernel(input_ids, table, offsets):
    ids = input_ids.astype(jnp.int32).reshape(TOTAL)
    offs16 = jnp.tile(offsets.astype(jnp.int32), L // H)
    table_pad = jnp.concatenate(
        [table, jnp.zeros((TOTAL_ROWS, DP - D), jnp.float32)], axis=1)
    out_pad = _sc_gather(ids, table_pad, offs16)
    return out_pad.reshape(B, S, H, DP)[..., :D]
